# baseline (speedup 1.0000x reference)
import jax
import jax.numpy as jnp
from jax.experimental import pallas as pl

B = 1024
IMG = 28
FEAT = 64
OUT = 10
_F32 = jnp.float32
_BF = jnp.bfloat16


def _fused(x1_ref, W_att_ref, Wcat_ref, bias_ref, Wfc_ref, out_ref):
    x1 = x1_ref[...]
    b_att = bias_ref[0:1, :]
    b_conv = bias_ref[1:2, :]
    gamma = bias_ref[2:3, :]
    beta = bias_ref[3:4, :]
    x2 = jnp.dot(x1, W_att_ref[...].astype(_BF),
                 preferred_element_type=_F32) + b_att

    x2b = x2.astype(_BF)
    Wcat = Wcat_ref[...]
    W_kq = (Wcat[:, :FEAT] + Wcat[:, FEAT:2 * FEAT]).astype(_BF)
    kq = jnp.dot(x2b, W_kq, preferred_element_type=_F32)
    v = jnp.dot(x2b, Wcat[:, 2 * FEAT:3 * FEAT].astype(_BF),
                preferred_element_type=_F32)
    skip = jnp.dot(x2b, Wcat[:, 3 * FEAT:].astype(_BF),
                   preferred_element_type=_F32)
    x4 = jax.nn.relu(skip + b_conv
                     + jnp.float32(B) * jax.nn.sigmoid(kq) * v)

    mean = jnp.mean(x4, axis=0, keepdims=True)
    var = jnp.mean((x4 - mean) ** 2, axis=0, keepdims=True)
    xn = (x4 - mean) * jax.lax.rsqrt(var + 1e-5) * gamma + beta

    r = jax.lax.broadcasted_iota(jnp.int32, (FEAT, FEAT // 2), 0)
    c = jax.lax.broadcasted_iota(jnp.int32, (FEAT, FEAT // 2), 1)
    s_even = (r == 2 * c).astype(_F32)
    s_odd = (r == 2 * c + 1).astype(_F32)
    xp = jnp.maximum(jnp.dot(xn, s_even, preferred_element_type=_F32),
                     jnp.dot(xn, s_odd, preferred_element_type=_F32))

    xpb = xp.astype(_BF)
    logits = jax.lax.dot_general(xpb, xpb, (((1,), (1,)), ((), ())),
                                 preferred_element_type=_F32)
    att = jax.nn.sigmoid(logits)
    rowsum = jnp.sum(att, axis=1, keepdims=True)
    x5 = jnp.dot(att.astype(_BF), xpb, preferred_element_type=_F32) / rowsum

    out_ref[...] = (jnp.dot(x5 + xp, Wfc_ref[:FEAT // 2, :],
                            preferred_element_type=_F32)
                    + Wfc_ref[FEAT // 2:FEAT // 2 + 1, :])


def kernel(x, train, W_att, b_att, W_key, W_query, W_value, W_skip, b_conv,
           gamma, beta, W_fc, b_fc):
    del train
    Bs = x.shape[0]
    x1 = x.reshape(Bs, IMG * IMG).astype(_BF)
    Wcat = jnp.concatenate([W_key, W_query, W_value, W_skip], axis=1)
    bias_cat = jnp.stack([b_att, b_conv, gamma, beta], axis=0)
    Wfc_cat = jnp.concatenate([W_fc, b_fc.reshape(1, OUT)], axis=0)
    return pl.pallas_call(
        _fused,
        out_shape=jax.ShapeDtypeStruct((Bs, OUT), _F32),
    )(x1, W_att, Wcat, bias_cat, Wfc_cat)


# speedup vs baseline: 1.1000x; 1.1000x over previous
import jax
import jax.numpy as jnp
from jax.experimental import pallas as pl

B = 1024
IMG = 28
FEAT = 64
OUT = 10
_F32 = jnp.float32


def _fused(x1_ref, W_att_ref, b_att_ref, W_key_ref, W_query_ref, W_value_ref,
           W_skip_ref, b_conv_ref, gamma_ref, beta_ref, W_fc_ref, b_fc_ref,
           out_ref):
    _BF = jnp.bfloat16
    x1 = x1_ref[...]
    x2 = jnp.dot(x1, W_att_ref[...].astype(_BF),
                 preferred_element_type=_F32) + b_att_ref[...]

    x2b = x2.astype(_BF)
    W_kq = (W_key_ref[...] + W_query_ref[...]).astype(_BF)
    kq = jnp.dot(x2b, W_kq, preferred_element_type=_F32)
    v = jnp.dot(x2b, W_value_ref[...].astype(_BF), preferred_element_type=_F32)
    skip = jnp.dot(x2b, W_skip_ref[...].astype(_BF), preferred_element_type=_F32)
    x4 = jax.nn.relu(skip + b_conv_ref[...]
                     + jnp.float32(B) * jax.nn.sigmoid(kq) * v)

    mean = jnp.mean(x4, axis=0, keepdims=True)
    var = jnp.mean((x4 - mean) ** 2, axis=0, keepdims=True)
    xn = (x4 - mean) * jax.lax.rsqrt(var + 1e-5) * gamma_ref[...] + beta_ref[...]

    r = jax.lax.broadcasted_iota(jnp.int32, (FEAT, FEAT // 2), 0)
    c = jax.lax.broadcasted_iota(jnp.int32, (FEAT, FEAT // 2), 1)
    s_even = (r == 2 * c).astype(_F32)
    s_odd = (r == 2 * c + 1).astype(_F32)
    xp = jnp.maximum(jnp.dot(xn, s_even, preferred_element_type=_F32),
                     jnp.dot(xn, s_odd, preferred_element_type=_F32))

    xpb = xp.astype(_BF)
    logits = jax.lax.dot_general(xpb, xpb, (((1,), (1,)), ((), ())),
                                 preferred_element_type=_F32)
    att = 0.5 * (1.0 + jnp.tanh(logits * 0.5))
    rowsum = jnp.sum(att, axis=1, keepdims=True)
    x5 = jnp.dot(att.astype(_BF), xpb, preferred_element_type=_F32) / rowsum

    out_ref[...] = (jnp.dot(x5 + xp, W_fc_ref[...], preferred_element_type=_F32)
                    + b_fc_ref[...])


def kernel(x, train, W_att, b_att, W_key, W_query, W_value, W_skip, b_conv,
           gamma, beta, W_fc, b_fc):
    del train
    Bs = x.shape[0]
    x1 = x.reshape(Bs, IMG * IMG).astype(jnp.bfloat16)
    return pl.pallas_call(
        _fused,
        out_shape=jax.ShapeDtypeStruct((Bs, OUT), _F32),
    )(x1, W_att, b_att.reshape(1, FEAT), W_key, W_query, W_value, W_skip,
      b_conv.reshape(1, FEAT), gamma.reshape(1, FEAT), beta.reshape(1, FEAT),
      W_fc, b_fc.reshape(1, OUT))
